# baseline (speedup 1.0000x reference)
import functools

import jax
import jax.numpy as jnp
from jax import lax
from jax.experimental import pallas as pl
from jax.experimental.pallas import tpu as pltpu
from jax.experimental.pallas import tpu_sc as plsc

NH = 6
NPT = 8
HD = 32
CC = 192
PW = 98
PP = PW * PW
NWORK = 32
CHUNK = 304
P = NWORK * CHUNK
QE = 9984
NG = NH * NPT
FAST_CID = 0


def _attn_body(qext_ref, wtap_ref, bias_ref, attn_ref):
    q = qext_ref[0]
    acc = jnp.zeros((NG, P), jnp.float32)
    for t in range(9):
        off = (t // 3) * PW + (t % 3)
        acc = acc + jnp.dot(wtap_ref[t], q[:, off:off + P],
                            preferred_element_type=jnp.float32)
    a3 = acc.reshape(NH, NPT, P) + bias_ref[...].reshape(NH, NPT, 1)
    m = jnp.max(a3, axis=1, keepdims=True)
    e = jnp.exp(a3 - m)
    attn = e / jnp.sum(e, axis=1, keepdims=True)
    attn_ref[0] = attn.reshape(NG, P)


def _attn_weights(qext, wtap, bias2):
    B = qext.shape[0]
    return pl.pallas_call(
        _attn_body,
        grid=(B,),
        in_specs=[
            pl.BlockSpec((1, CC, QE), lambda b: (b, 0, 0)),
            pl.BlockSpec((9, NG, CC), lambda b: (0, 0, 0)),
            pl.BlockSpec((NG, 1), lambda b: (0, 0)),
        ],
        out_specs=pl.BlockSpec((1, NG, P), lambda b: (b, 0, 0)),
        out_shape=jax.ShapeDtypeStruct((B, NG, P), jnp.float32),
    )(qext, wtap, bias2)


_GDN = lax.GatherDimensionNumbers(offset_dims=(), collapsed_slice_dims=(0,),
                                  start_index_map=(0,))


def _bcast(vec, i):
    return lax.gather(vec, jnp.full((16, 1), i, jnp.int32), _GDN, (1,),
                      mode=lax.GatherScatterMode.PROMISE_IN_BOUNDS)


def _sc_gather(table, attn, gx, gy, S):
    f32 = jnp.float32
    i32 = jnp.int32
    mesh = plsc.VectorSubcoreMesh(core_axis_name="c", subcore_axis_name="s")
    NSETS = 4

    def scr():
        return ([pltpu.VMEM((CHUNK,), i32) for _ in range(2)]
                + [pltpu.VMEM((CHUNK, HD), i32) for _ in range(2)]
                + [pltpu.VMEM((CHUNK,), f32) for _ in range(7)]
                + [pltpu.SemaphoreType.DMA, pltpu.SemaphoreType.DMA])

    NREF = 13
    NCH = P // CHUNK

    @functools.partial(
        pl.kernel, mesh=mesh,
        compiler_params=pltpu.CompilerParams(use_tc_tiling_on_sc=False),
        out_type=jax.ShapeDtypeStruct((S * P, HD), f32),
        scratch_types=scr() * NSETS + [pltpu.VMEM((CHUNK, HD), f32)],
    )
    def k(table_hbm, attn_hbm, gx_hbm, gy_hbm, out_hbm, *refs):
        sets = [refs[i * NREF:(i + 1) * NREF] for i in range(NSETS)]
        acc = refs[NSETS * NREF]
        cid = lax.axis_index("c")
        sid = lax.axis_index("s")
        icps = [None] * NSETS
        gcps = [None] * NSETS

        wid = cid * 16 + sid
        u0 = wid * 12
        cnt = jnp.int32(12)

        def offs(step):
            u = u0 + step // NPT
            p = step % NPT
            s = u // NCH
            ch = lax.rem(u, NCH)
            pix0 = ch * CHUNK
            return s, pix0, (s * NPT + p) * P + pix0

        def fetch_idx(step_r, step):
            st = step % NSETS
            gxv, gyv, av = sets[st][8:11]
            semi = sets[st][11]
            _s, _pix0, foff = offs(step_r + step)

            @pl.when(step_r + step < cnt * NPT)
            def _():
                icps[st] = [
                    pltpu.async_copy(gx_hbm.at[pl.ds(foff, CHUNK)], gxv, semi),
                    pltpu.async_copy(gy_hbm.at[pl.ds(foff, CHUNK)], gyv, semi),
                    pltpu.async_copy(attn_hbm.at[pl.ds(foff, CHUNK)], av, semi),
                ]

        def fire(step_r, step):
            st = step % NSETS
            i0, i2 = sets[st][0:2]
            g0, g2 = sets[st][2:4]
            v0, v1, v2, v3 = sets[st][4:8]
            gxv, gyv, av = sets[st][8:11]
            semg = sets[st][12]
            s, _pix0, _foff = offs(step_r + step)
            sbase = s * PP

            @pl.when(step_r + step < cnt * NPT)
            def _():
                for c in icps[st]:
                    c.wait()

                @plsc.parallel_loop(0, CHUNK // 16, 1)
                def dbody(j):
                    sl = pl.ds(j * 16, 16)
                    x1 = gxv[sl] * 96.0 + 0.5
                    y1 = gyv[sl] * 96.0 + 0.5
                    a16 = av[sl]
                    tx = x1.astype(i32)
                    ty = y1.astype(i32)
                    wx1 = x1 - tx.astype(f32)
                    wx0 = 1.0 - wx1
                    wy1 = y1 - ty.astype(f32)
                    wy0 = 1.0 - wy1
                    aw0 = a16 * wy0
                    aw1 = a16 * wy1
                    v0[sl] = aw0 * wx0
                    v1[sl] = aw0 * wx1
                    v2[sl] = aw1 * wx0
                    v3[sl] = aw1 * wx1
                    base = ty * PW + tx + sbase
                    i0[sl] = base
                    i2[sl] = base + PW
                gcps[st] = [
                    pltpu.async_copy(table_hbm.at[i0], g0, semg),
                    pltpu.async_copy(table_hbm.at[i2], g2, semg),
                ]

        mhi = jnp.int32(-65536)

        def compute(p):
            st = p % NSETS
            g0, g2 = sets[st][2:4]
            v0, v1, v2, v3 = sets[st][4:8]
            first = p == 0

            @plsc.parallel_loop(0, CHUNK // 16, 1)
            def cbody(gi):
                base = gi * 16
                wv0 = v0[pl.ds(base, 16)]
                wv1 = v1[pl.ds(base, 16)]
                wv2 = v2[pl.ds(base, 16)]
                wv3 = v3[pl.ds(base, 16)]
                for i in range(16):
                    px = base + i
                    bw = [_bcast(wv0, i), _bcast(wv1, i),
                          _bcast(wv2, i), _bcast(wv3, i)]
                    xs = [g0[px, pl.ds(0, 16)], g0[px, pl.ds(16, 16)],
                          g2[px, pl.ds(0, 16)], g2[px, pl.ds(16, 16)]]
                    plos, phis = [], []
                    for b, x in zip(bw, xs):
                        xlo = lax.bitcast_convert_type(
                            lax.shift_left(x, 16), f32)
                        xhi = lax.bitcast_convert_type(x & mhi, f32)
                        plos.append(b * xlo)
                        phis.append(b * xhi)
                    slo = (plos[0] + plos[1]) + (plos[2] + plos[3])
                    shi = (phis[0] + phis[1]) + (phis[2] + phis[3])
                    if not first:
                        slo = slo + acc[px, pl.ds(0, 16)]
                        shi = shi + acc[px, pl.ds(16, 16)]
                    acc[px, pl.ds(0, 16)] = slo
                    acc[px, pl.ds(16, 16)] = shi

        def unit_body(ul, carry):
            step_r = ul * NPT
            for p in range(NPT):
                fetch_idx(step_r, p + 2)
                fire(step_r, p + 1)
                for c in gcps[p % NSETS]:
                    c.wait()
                compute(p)
            s, pix0, _f = offs(step_r)
            pltpu.sync_copy(acc, out_hbm.at[pl.ds(s * P + pix0, CHUNK)])
            return carry

        fetch_idx(0, 0)
        fire(0, 0)
        fetch_idx(0, 1)
        lax.fori_loop(0, cnt, unit_body, 0)

    return k(table, attn, gx, gy)


def _proj_body(pw_ref, ws_ref, bias_ref, out_ref, acc_ref, wsb_ref, sem):
    b = pl.program_id(0)
    h = pl.program_id(1)
    s = b * NH + h
    cp = pltpu.make_async_copy(ws_ref.at[pl.ds(s * P, P)], wsb_ref, sem)
    cp.start()
    cp.wait()
    res = lax.dot_general(pw_ref[0], wsb_ref[...],
                          (((1,), (1,)), ((), ())),
                          preferred_element_type=jnp.float32)

    @pl.when(h == 0)
    def _():
        acc_ref[...] = res + bias_ref[...]

    @pl.when(h > 0)
    def _():
        acc_ref[...] = acc_ref[...] + res

    @pl.when(h == NH - 1)
    def _():
        for y in range(96):
            out_ref[0, :, y, :] = acc_ref[:, y * PW:y * PW + 96]


def _proj(pw2, ws2, pb2, B, H, W):
    return pl.pallas_call(
        _proj_body,
        grid=(B, NH),
        in_specs=[
            pl.BlockSpec((1, CC, HD), lambda b, h: (h, 0, 0)),
            pl.BlockSpec(memory_space=pl.ANY),
            pl.BlockSpec((CC, 1), lambda b, h: (0, 0)),
        ],
        out_specs=pl.BlockSpec((1, CC, H, W), lambda b, h: (b, 0, 0, 0)),
        out_shape=jax.ShapeDtypeStruct((B, CC, H, W), jnp.float32),
        scratch_shapes=[pltpu.VMEM((CC, P), jnp.float32),
                        pltpu.VMEM((P, HD), jnp.float32),
                        pltpu.SemaphoreType.DMA],
        compiler_params=pltpu.CompilerParams(
            dimension_semantics=("parallel", "arbitrary")),
    )(pw2, ws2, pb2)


def kernel(query, value, reference_points, attn_conv_w, attn_conv_b,
           proj_w, proj_b):
    B, C, H, W = query.shape

    qpad = jnp.pad(query.astype(jnp.bfloat16), ((0, 0), (0, 0), (1, 1), (1, 1)))
    qext = jnp.pad(qpad.reshape(B, C, PP), ((0, 0), (0, 0), (0, QE - PP)))

    wtap = attn_conv_w.reshape(NG, C, 9).transpose(2, 0, 1) \
                      .astype(jnp.bfloat16)
    bias2 = attn_conv_b.reshape(NG, 1)

    vb = value.astype(jnp.bfloat16).reshape(B, NH, 2, HD // 2, H, W)
    vt = vb.transpose(0, 1, 4, 5, 3, 2)
    vw = lax.bitcast_convert_type(vt, jnp.int32)
    tw = jnp.pad(vw, ((0, 0), (0, 0), (1, 1), (1, 1), (0, 0))) \
            .reshape(B * NH * PP, HD // 2)
    twp = jnp.pad(tw, ((0, 1), (0, 0)))
    table = jnp.concatenate([twp[:-1], twp[1:]], axis=1)

    rpt = reference_points.transpose(0, 3, 4, 5, 1, 2)
    rpp = jnp.pad(rpt, ((0, 0), (0, 0), (0, 0), (0, 0), (0, 2), (0, 2)),
                  constant_values=0.5)
    gx = jnp.pad(rpp[:, :, :, 0].reshape(B, NG, PP),
                 ((0, 0), (0, 0), (0, P - PP)), constant_values=0.5)
    gy = jnp.pad(rpp[:, :, :, 1].reshape(B, NG, PP),
                 ((0, 0), (0, 0), (0, P - PP)), constant_values=0.5)

    attn = _attn_weights(qext, wtap, bias2)

    S = B * NH
    ws = _sc_gather(table, attn.reshape(-1), gx.reshape(-1), gy.reshape(-1),
                    S)

    pw2 = proj_w.reshape(C, NH, HD).transpose(1, 0, 2)
    pb2 = proj_b.reshape(C, 1)
    return _proj(pw2, ws, pb2, B, H, W)


# speedup vs baseline: 1.1589x; 1.1589x over previous
import functools

import jax
import jax.numpy as jnp
from jax import lax
from jax.experimental import pallas as pl
from jax.experimental.pallas import tpu as pltpu
from jax.experimental.pallas import tpu_sc as plsc

NH = 6
NPT = 8
HD = 32
CC = 192
PW = 98
PP = PW * PW
NWORK = 32
CHUNK = 304
P = NWORK * CHUNK
QE = 9984
NG = NH * NPT
FAST_CID = 0


def _attn_body(qext_ref, wtap_ref, bias_ref, attn_ref):
    q = qext_ref[0]
    acc = jnp.zeros((NG, P), jnp.float32)
    for t in range(9):
        off = (t // 3) * PW + (t % 3)
        acc = acc + jnp.dot(wtap_ref[t], q[:, off:off + P],
                            preferred_element_type=jnp.float32)
    a3 = acc.reshape(NH, NPT, P) + bias_ref[...].reshape(NH, NPT, 1)
    m = jnp.max(a3, axis=1, keepdims=True)
    e = jnp.exp(a3 - m)
    attn = e / jnp.sum(e, axis=1, keepdims=True)
    attn_ref[0] = attn.reshape(NG, P)


def _attn_weights(qext, wtap, bias2):
    B = qext.shape[0]
    return pl.pallas_call(
        _attn_body,
        grid=(B,),
        in_specs=[
            pl.BlockSpec((1, CC, QE), lambda b: (b, 0, 0)),
            pl.BlockSpec((9, NG, CC), lambda b: (0, 0, 0)),
            pl.BlockSpec((NG, 1), lambda b: (0, 0)),
        ],
        out_specs=pl.BlockSpec((1, NG, P), lambda b: (b, 0, 0)),
        out_shape=jax.ShapeDtypeStruct((B, NG, P), jnp.float32),
    )(qext, wtap, bias2)


_GDN = lax.GatherDimensionNumbers(offset_dims=(), collapsed_slice_dims=(0,),
                                  start_index_map=(0,))


def _bcast(vec, i):
    return lax.gather(vec, jnp.full((16, 1), i, jnp.int32), _GDN, (1,),
                      mode=lax.GatherScatterMode.PROMISE_IN_BOUNDS)


def _sc_gather(table, attn, gx, gy, S):
    f32 = jnp.float32
    i32 = jnp.int32
    mesh = plsc.VectorSubcoreMesh(core_axis_name="c", subcore_axis_name="s")
    NSETS = 4

    def scr():
        return ([pltpu.VMEM((CHUNK,), i32) for _ in range(2)]
                + [pltpu.VMEM((CHUNK, HD), i32) for _ in range(2)]
                + [pltpu.VMEM((CHUNK,), f32) for _ in range(7)]
                + [pltpu.SemaphoreType.DMA, pltpu.SemaphoreType.DMA])

    NREF = 13
    NCH = P // CHUNK

    @functools.partial(
        pl.kernel, mesh=mesh,
        compiler_params=pltpu.CompilerParams(use_tc_tiling_on_sc=False),
        out_type=jax.ShapeDtypeStruct((S * P * HD,), f32),
        scratch_types=scr() * NSETS + [pltpu.VMEM((CHUNK * HD,), f32)],
    )
    def k(table_hbm, attn_hbm, gx_hbm, gy_hbm, out_hbm, *refs):
        sets = [refs[i * NREF:(i + 1) * NREF] for i in range(NSETS)]
        acc = refs[NSETS * NREF]
        cid = lax.axis_index("c")
        sid = lax.axis_index("s")
        icps = [None] * NSETS
        gcps = [None] * NSETS

        wid = cid * 16 + sid
        u0 = wid * 12
        cnt = jnp.int32(12)

        def offs(step):
            u = u0 + step // NPT
            p = step % NPT
            s = u // NCH
            ch = lax.rem(u, NCH)
            pix0 = ch * CHUNK
            return s, pix0, (s * NPT + p) * P + pix0

        def fetch_idx(step_r, step):
            st = step % NSETS
            gxv, gyv, av = sets[st][8:11]
            semi = sets[st][11]
            _s, _pix0, foff = offs(step_r + step)

            @pl.when(step_r + step < cnt * NPT)
            def _():
                icps[st] = [
                    pltpu.async_copy(gx_hbm.at[pl.ds(foff, CHUNK)], gxv, semi),
                    pltpu.async_copy(gy_hbm.at[pl.ds(foff, CHUNK)], gyv, semi),
                    pltpu.async_copy(attn_hbm.at[pl.ds(foff, CHUNK)], av, semi),
                ]

        def fire(step_r, step):
            st = step % NSETS
            i0, i2 = sets[st][0:2]
            g0, g2 = sets[st][2:4]
            v0, v1, v2, v3 = sets[st][4:8]
            gxv, gyv, av = sets[st][8:11]
            semg = sets[st][12]
            s, _pix0, _foff = offs(step_r + step)
            sbase = s * PP

            @pl.when(step_r + step < cnt * NPT)
            def _():
                for c in icps[st]:
                    c.wait()

                @plsc.parallel_loop(0, CHUNK // 16, 1)
                def dbody(j):
                    sl = pl.ds(j * 16, 16)
                    x1 = gxv[sl] * 96.0 + 0.5
                    y1 = gyv[sl] * 96.0 + 0.5
                    a16 = av[sl]
                    tx = x1.astype(i32)
                    ty = y1.astype(i32)
                    wx1 = x1 - tx.astype(f32)
                    wx0 = 1.0 - wx1
                    wy1 = y1 - ty.astype(f32)
                    wy0 = 1.0 - wy1
                    aw0 = a16 * wy0
                    aw1 = a16 * wy1
                    v0[sl] = aw0 * wx0
                    v1[sl] = aw0 * wx1
                    v2[sl] = aw1 * wx0
                    v3[sl] = aw1 * wx1
                    base = ty * PW + tx + sbase
                    i0[sl] = base
                    i2[sl] = base + PW
                gcps[st] = [
                    pltpu.async_copy(table_hbm.at[i0], g0, semg),
                    pltpu.async_copy(table_hbm.at[i2], g2, semg),
                ]

        def compute(p):
            st = p % NSETS
            g0, g2 = sets[st][2:4]
            v0, v1, v2, v3 = sets[st][4:8]
            first = p == 0

            @plsc.parallel_loop(0, CHUNK // 16, 1)
            def cbody(gi):
                base = gi * 16
                wv0 = v0[pl.ds(base, 16)]
                wv1 = v1[pl.ds(base, 16)]
                wv2 = v2[pl.ds(base, 16)]
                wv3 = v3[pl.ds(base, 16)]
                for i in range(16):
                    px = base + i
                    bw = [_bcast(wv0, i), _bcast(wv1, i),
                          _bcast(wv2, i), _bcast(wv3, i)]
                    xs = [g0[px, pl.ds(0, 16)], g0[px, pl.ds(16, 16)],
                          g2[px, pl.ds(0, 16)], g2[px, pl.ds(16, 16)]]
                    plos, phis = [], []
                    for b, x in zip(bw, xs):
                        xlo = lax.bitcast_convert_type(
                            lax.shift_left(x, 16), f32)
                        xhi = lax.bitcast_convert_type(x, f32)
                        plos.append(b * xlo)
                        phis.append(b * xhi)
                    slo = (plos[0] + plos[1]) + (plos[2] + plos[3])
                    shi = (phis[0] + phis[1]) + (phis[2] + phis[3])
                    if not first:
                        slo = slo + acc[pl.ds(px * HD, 16)]
                        shi = shi + acc[pl.ds(px * HD + 16, 16)]
                    acc[pl.ds(px * HD, 16)] = slo
                    acc[pl.ds(px * HD + 16, 16)] = shi

        def unit_body(ul, carry):
            step_r = ul * NPT
            for p in range(NPT):
                fetch_idx(step_r, p + 2)
                fire(step_r, p + 1)
                for c in gcps[p % NSETS]:
                    c.wait()
                compute(p)
            s, pix0, _f = offs(step_r)
            pltpu.sync_copy(acc, out_hbm.at[pl.ds((s * P + pix0) * HD,
                                                  CHUNK * HD)])
            return carry

        fetch_idx(0, 0)
        fire(0, 0)
        fetch_idx(0, 1)
        lax.fori_loop(0, cnt, unit_body, 0)

    return k(table, attn, gx, gy)


def _proj_body(pw_ref, ws_ref, bias_ref, out_ref):
    h = pl.program_id(1)
    res = lax.dot_general(pw_ref[0], ws_ref[0, 0],
                          (((1,), (1,)), ((), ())),
                          preferred_element_type=jnp.float32)

    @pl.when(h == 0)
    def _():
        out_ref[0] = res + bias_ref[...]

    @pl.when(h > 0)
    def _():
        out_ref[0] = out_ref[0] + res


def _proj(pw2, ws4, pb2):
    B = ws4.shape[0]
    return pl.pallas_call(
        _proj_body,
        grid=(B, NH),
        in_specs=[
            pl.BlockSpec((1, CC, HD), lambda b, h: (h, 0, 0)),
            pl.BlockSpec((1, 1, P, HD), lambda b, h: (b, h, 0, 0)),
            pl.BlockSpec((CC, 1), lambda b, h: (0, 0)),
        ],
        out_specs=pl.BlockSpec((1, CC, P), lambda b, h: (b, 0, 0)),
        out_shape=jax.ShapeDtypeStruct((B, CC, P), jnp.float32),
        compiler_params=pltpu.CompilerParams(
            dimension_semantics=("parallel", "arbitrary")),
    )(pw2, ws4, pb2)


def kernel(query, value, reference_points, attn_conv_w, attn_conv_b,
           proj_w, proj_b):
    B, C, H, W = query.shape

    qpad = jnp.pad(query.astype(jnp.bfloat16), ((0, 0), (0, 0), (1, 1), (1, 1)))
    qext = jnp.pad(qpad.reshape(B, C, PP), ((0, 0), (0, 0), (0, QE - PP)))

    wtap = attn_conv_w.reshape(NG, C, 9).transpose(2, 0, 1) \
                      .astype(jnp.bfloat16)
    bias2 = attn_conv_b.reshape(NG, 1)

    vb = value.astype(jnp.bfloat16).reshape(B, NH, 2, HD // 2, H, W)
    vt = vb.transpose(0, 1, 4, 5, 3, 2)
    vw = lax.bitcast_convert_type(vt, jnp.int32)
    tw = jnp.pad(vw, ((0, 0), (0, 0), (1, 1), (1, 1), (0, 0))) \
            .reshape(B * NH * PP, HD // 2)
    twp = jnp.pad(tw, ((0, 1), (0, 0)))
    table = jnp.concatenate([twp[:-1], twp[1:]], axis=1)

    rpt = reference_points.transpose(0, 3, 4, 5, 1, 2)
    rpp = jnp.pad(rpt, ((0, 0), (0, 0), (0, 0), (0, 0), (0, 2), (0, 2)),
                  constant_values=0.5)
    gx = jnp.pad(rpp[:, :, :, 0].reshape(B, NG, PP),
                 ((0, 0), (0, 0), (0, P - PP)), constant_values=0.5)
    gy = jnp.pad(rpp[:, :, :, 1].reshape(B, NG, PP),
                 ((0, 0), (0, 0), (0, P - PP)), constant_values=0.5)

    attn = _attn_weights(qext, wtap, bias2)

    S = B * NH
    ws = _sc_gather(table, attn.reshape(-1), gx.reshape(-1), gy.reshape(-1),
                    S)

    pw2 = proj_w.reshape(C, NH, HD).transpose(1, 0, 2)
    pb2 = proj_b.reshape(C, 1)
    outflat = _proj(pw2, ws.reshape(B, NH, P, HD), pb2)
    return outflat[:, :, :PP].reshape(B, C, PW, PW)[:, :, :H, :W]


# speedup vs baseline: 1.2107x; 1.0447x over previous
import functools

import jax
import jax.numpy as jnp
from jax import lax
from jax.experimental import pallas as pl
from jax.experimental.pallas import tpu as pltpu
from jax.experimental.pallas import tpu_sc as plsc

NH = 6
NPT = 8
HD = 32
CC = 192
PW = 98
PP = PW * PW
NWORK = 32
P = 9728
CHUNK = 608
QE = 9984
NG = NH * NPT
FAST_CID = 0


def _attn_body(qext_ref, wtap_ref, bias_ref, attn_ref):
    q = qext_ref[0]
    acc = jnp.zeros((NG, P), jnp.float32)
    for t in range(9):
        off = (t // 3) * PW + (t % 3)
        acc = acc + jnp.dot(wtap_ref[t], q[:, off:off + P],
                            preferred_element_type=jnp.float32)
    a3 = acc.reshape(NH, NPT, P) + bias_ref[...].reshape(NH, NPT, 1)
    m = jnp.max(a3, axis=1, keepdims=True)
    e = jnp.exp(a3 - m)
    attn = e / jnp.sum(e, axis=1, keepdims=True)
    attn_ref[0] = attn.reshape(NG, P)


def _attn_weights(qext, wtap, bias2):
    B = qext.shape[0]
    return pl.pallas_call(
        _attn_body,
        grid=(B,),
        in_specs=[
            pl.BlockSpec((1, CC, QE), lambda b: (b, 0, 0)),
            pl.BlockSpec((9, NG, CC), lambda b: (0, 0, 0)),
            pl.BlockSpec((NG, 1), lambda b: (0, 0)),
        ],
        out_specs=pl.BlockSpec((1, NG, P), lambda b: (b, 0, 0)),
        out_shape=jax.ShapeDtypeStruct((B, NG, P), jnp.float32),
    )(qext, wtap, bias2)


_GDN = lax.GatherDimensionNumbers(offset_dims=(), collapsed_slice_dims=(0,),
                                  start_index_map=(0,))


def _bcast(vec, i):
    return lax.gather(vec, jnp.full((16, 1), i, jnp.int32), _GDN, (1,),
                      mode=lax.GatherScatterMode.PROMISE_IN_BOUNDS)


def _sc_gather(table, attn, gx, gy, S):
    f32 = jnp.float32
    i32 = jnp.int32
    mesh = plsc.VectorSubcoreMesh(core_axis_name="c", subcore_axis_name="s")
    NSETS = 2

    def scr():
        return ([pltpu.VMEM((CHUNK,), i32) for _ in range(2)]
                + [pltpu.VMEM((CHUNK, HD), i32) for _ in range(2)]
                + [pltpu.VMEM((CHUNK,), f32) for _ in range(7)]
                + [pltpu.SemaphoreType.DMA, pltpu.SemaphoreType.DMA])

    NREF = 13
    NCH = P // CHUNK

    @functools.partial(
        pl.kernel, mesh=mesh,
        compiler_params=pltpu.CompilerParams(use_tc_tiling_on_sc=False),
        out_type=jax.ShapeDtypeStruct((S * P * HD,), f32),
        scratch_types=scr() * NSETS + [pltpu.VMEM((CHUNK * HD,), f32)],
    )
    def k(table_hbm, attn_hbm, gx_hbm, gy_hbm, out_hbm, *refs):
        sets = [refs[i * NREF:(i + 1) * NREF] for i in range(NSETS)]
        acc = refs[NSETS * NREF]
        cid = lax.axis_index("c")
        sid = lax.axis_index("s")
        icps = [None] * NSETS
        gcps = [None] * NSETS

        wid = cid * 16 + sid
        u0 = wid * 6
        cnt = jnp.int32(6)

        def offs(step):
            u = u0 + step // NPT
            p = step % NPT
            s = u // NCH
            ch = lax.rem(u, NCH)
            pix0 = ch * CHUNK
            return s, pix0, (s * NPT + p) * P + pix0

        def fetch_idx(step_r, step):
            st = step % NSETS
            gxv, gyv, av = sets[st][8:11]
            semi = sets[st][11]
            _s, _pix0, foff = offs(step_r + step)

            @pl.when(step_r + step < cnt * NPT)
            def _():
                icps[st] = [
                    pltpu.async_copy(gx_hbm.at[pl.ds(foff, CHUNK)], gxv, semi),
                    pltpu.async_copy(gy_hbm.at[pl.ds(foff, CHUNK)], gyv, semi),
                    pltpu.async_copy(attn_hbm.at[pl.ds(foff, CHUNK)], av, semi),
                ]

        def fire(step_r, step):
            st = step % NSETS
            i0, i2 = sets[st][0:2]
            g0, g2 = sets[st][2:4]
            v0, v1, v2, v3 = sets[st][4:8]
            gxv, gyv, av = sets[st][8:11]
            semg = sets[st][12]
            s, _pix0, _foff = offs(step_r + step)
            sbase = s * PP

            @pl.when(step_r + step < cnt * NPT)
            def _():
                for c in icps[st]:
                    c.wait()

                @plsc.parallel_loop(0, CHUNK // 16, 1)
                def dbody(j):
                    sl = pl.ds(j * 16, 16)
                    x1 = gxv[sl] * 96.0 + 0.5
                    y1 = gyv[sl] * 96.0 + 0.5
                    a16 = av[sl]
                    tx = x1.astype(i32)
                    ty = y1.astype(i32)
                    wx1 = x1 - tx.astype(f32)
                    wx0 = 1.0 - wx1
                    wy1 = y1 - ty.astype(f32)
                    wy0 = 1.0 - wy1
                    aw0 = a16 * wy0
                    aw1 = a16 * wy1
                    v0[sl] = aw0 * wx0
                    v1[sl] = aw0 * wx1
                    v2[sl] = aw1 * wx0
                    v3[sl] = aw1 * wx1
                    base = ty * PW + tx + sbase
                    i0[sl] = base
                    i2[sl] = base + PW
                gcps[st] = [
                    pltpu.async_copy(table_hbm.at[i0], g0, semg),
                    pltpu.async_copy(table_hbm.at[i2], g2, semg),
                ]

        def compute(p):
            st = p % NSETS
            g0, g2 = sets[st][2:4]
            v0, v1, v2, v3 = sets[st][4:8]
            first = p == 0

            @plsc.parallel_loop(0, CHUNK // 16, 1)
            def cbody(gi):
                base = gi * 16
                wv0 = v0[pl.ds(base, 16)]
                wv1 = v1[pl.ds(base, 16)]
                wv2 = v2[pl.ds(base, 16)]
                wv3 = v3[pl.ds(base, 16)]
                for i in range(16):
                    px = base + i
                    bw = [_bcast(wv0, i), _bcast(wv1, i),
                          _bcast(wv2, i), _bcast(wv3, i)]
                    xs = [g0[px, pl.ds(0, 16)], g0[px, pl.ds(16, 16)],
                          g2[px, pl.ds(0, 16)], g2[px, pl.ds(16, 16)]]
                    plos, phis = [], []
                    for b, x in zip(bw, xs):
                        xlo = lax.bitcast_convert_type(
                            lax.shift_left(x, 16), f32)
                        xhi = lax.bitcast_convert_type(x, f32)
                        plos.append(b * xlo)
                        phis.append(b * xhi)
                    slo = (plos[0] + plos[1]) + (plos[2] + plos[3])
                    shi = (phis[0] + phis[1]) + (phis[2] + phis[3])
                    if not first:
                        slo = slo + acc[pl.ds(px * HD, 16)]
                        shi = shi + acc[pl.ds(px * HD + 16, 16)]
                    acc[pl.ds(px * HD, 16)] = slo
                    acc[pl.ds(px * HD + 16, 16)] = shi

        def unit_body(ul, carry):
            step_r = ul * NPT
            for p in range(NPT):
                fetch_idx(step_r, p + 2)
                fire(step_r, p + 1)
                for c in gcps[p % NSETS]:
                    c.wait()
                compute(p)
            s, pix0, _f = offs(step_r)
            pltpu.sync_copy(acc, out_hbm.at[pl.ds((s * P + pix0) * HD,
                                                  CHUNK * HD)])
            return carry

        fetch_idx(0, 0)
        fire(0, 0)
        fetch_idx(0, 1)
        lax.fori_loop(0, cnt, unit_body, 0)

    return k(table, attn, gx, gy)


def _proj_body(pw_ref, ws_ref, bias_ref, out_ref):
    h = pl.program_id(1)
    res = lax.dot_general(pw_ref[0], ws_ref[0, 0],
                          (((1,), (1,)), ((), ())),
                          preferred_element_type=jnp.float32)

    @pl.when(h == 0)
    def _():
        out_ref[0] = res + bias_ref[...]

    @pl.when(h > 0)
    def _():
        out_ref[0] = out_ref[0] + res


def _proj(pw2, ws4, pb2):
    B = ws4.shape[0]
    return pl.pallas_call(
        _proj_body,
        grid=(B, NH),
        in_specs=[
            pl.BlockSpec((1, CC, HD), lambda b, h: (h, 0, 0)),
            pl.BlockSpec((1, 1, P, HD), lambda b, h: (b, h, 0, 0)),
            pl.BlockSpec((CC, 1), lambda b, h: (0, 0)),
        ],
        out_specs=pl.BlockSpec((1, CC, P), lambda b, h: (b, 0, 0)),
        out_shape=jax.ShapeDtypeStruct((B, CC, P), jnp.float32),
        compiler_params=pltpu.CompilerParams(
            dimension_semantics=("parallel", "arbitrary")),
    )(pw2, ws4, pb2)


def kernel(query, value, reference_points, attn_conv_w, attn_conv_b,
           proj_w, proj_b):
    B, C, H, W = query.shape

    qpad = jnp.pad(query.astype(jnp.bfloat16), ((0, 0), (0, 0), (1, 1), (1, 1)))
    qext = jnp.pad(qpad.reshape(B, C, PP), ((0, 0), (0, 0), (0, QE - PP)))

    wtap = attn_conv_w.reshape(NG, C, 9).transpose(2, 0, 1) \
                      .astype(jnp.bfloat16)
    bias2 = attn_conv_b.reshape(NG, 1)

    vb = value.astype(jnp.bfloat16).reshape(B, NH, 2, HD // 2, H, W)
    vt = vb.transpose(0, 1, 4, 5, 3, 2)
    vw = lax.bitcast_convert_type(vt, jnp.int32)
    tw = jnp.pad(vw, ((0, 0), (0, 0), (1, 1), (1, 1), (0, 0))) \
            .reshape(B * NH * PP, HD // 2)
    twp = jnp.pad(tw, ((0, 1), (0, 0)))
    table = jnp.concatenate([twp[:-1], twp[1:]], axis=1)

    rpt = reference_points.transpose(0, 3, 4, 5, 1, 2)
    rpp = jnp.pad(rpt, ((0, 0), (0, 0), (0, 0), (0, 0), (0, 2), (0, 2)),
                  constant_values=0.5)
    gx = jnp.pad(rpp[:, :, :, 0].reshape(B, NG, PP),
                 ((0, 0), (0, 0), (0, P - PP)), constant_values=0.5)
    gy = jnp.pad(rpp[:, :, :, 1].reshape(B, NG, PP),
                 ((0, 0), (0, 0), (0, P - PP)), constant_values=0.5)

    attn = _attn_weights(qext, wtap, bias2)

    S = B * NH
    ws = _sc_gather(table, attn.reshape(-1), gx.reshape(-1), gy.reshape(-1),
                    S)

    pw2 = proj_w.reshape(C, NH, HD).transpose(1, 0, 2)
    pb2 = proj_b.reshape(C, 1)
    outflat = _proj(pw2, ws.reshape(B, NH, P, HD), pb2)
    return outflat[:, :, :PP].reshape(B, C, PW, PW)[:, :, :H, :W]


# speedup vs baseline: 1.2109x; 1.0002x over previous
import functools

import jax
import jax.numpy as jnp
from jax import lax
from jax.experimental import pallas as pl
from jax.experimental.pallas import tpu as pltpu
from jax.experimental.pallas import tpu_sc as plsc

NH = 6
NPT = 8
HD = 32
CC = 192
PW = 98
PP = PW * PW
P = 9728
CHUNK = 608
QE = 9984
NG = NH * NPT


def _attn_body(qext_ref, wtap_ref, bias_ref, attn_ref):
    q = qext_ref[0]
    acc = jnp.zeros((NG, P), jnp.float32)
    for t in range(9):
        off = (t // 3) * PW + (t % 3)
        acc = acc + jnp.dot(wtap_ref[t], q[:, off:off + P],
                            preferred_element_type=jnp.float32)
    a3 = acc.reshape(NH, NPT, P) + bias_ref[...].reshape(NH, NPT, 1)
    m = jnp.max(a3, axis=1, keepdims=True)
    e = jnp.exp(a3 - m)
    attn = e / jnp.sum(e, axis=1, keepdims=True)
    attn_ref[0] = attn.reshape(NG, P)


def _attn_weights(qext, wtap, bias2):
    B = qext.shape[0]
    return pl.pallas_call(
        _attn_body,
        grid=(B,),
        in_specs=[
            pl.BlockSpec((1, CC, QE), lambda b: (b, 0, 0)),
            pl.BlockSpec((9, NG, CC), lambda b: (0, 0, 0)),
            pl.BlockSpec((NG, 1), lambda b: (0, 0)),
        ],
        out_specs=pl.BlockSpec((1, NG, P), lambda b: (b, 0, 0)),
        out_shape=jax.ShapeDtypeStruct((B, NG, P), jnp.float32),
    )(qext, wtap, bias2)


_GDN = lax.GatherDimensionNumbers(offset_dims=(), collapsed_slice_dims=(0,),
                                  start_index_map=(0,))


def _bcast(vec, i):
    return lax.gather(vec, jnp.full((16, 1), i, jnp.int32), _GDN, (1,),
                      mode=lax.GatherScatterMode.PROMISE_IN_BOUNDS)


def _sc_gather(table, attn, gx, gy, S):
    f32 = jnp.float32
    i32 = jnp.int32
    mesh = plsc.VectorSubcoreMesh(core_axis_name="c", subcore_axis_name="s")
    NSETS = 2

    def scr():
        return ([pltpu.VMEM((CHUNK,), i32) for _ in range(2)]
                + [pltpu.VMEM((CHUNK, HD), i32) for _ in range(2)]
                + [pltpu.VMEM((CHUNK,), f32) for _ in range(7)]
                + [pltpu.SemaphoreType.DMA, pltpu.SemaphoreType.DMA])

    NREF = 13
    NCH = P // CHUNK

    @functools.partial(
        pl.kernel, mesh=mesh,
        compiler_params=pltpu.CompilerParams(use_tc_tiling_on_sc=False),
        out_type=jax.ShapeDtypeStruct((S * P * HD,), f32),
        scratch_types=scr() * NSETS + [pltpu.VMEM((CHUNK * HD,), f32)],
    )
    def k(table_hbm, attn_hbm, gx_hbm, gy_hbm, out_hbm, *refs):
        sets = [refs[i * NREF:(i + 1) * NREF] for i in range(NSETS)]
        acc = refs[NSETS * NREF]
        cid = lax.axis_index("c")
        sid = lax.axis_index("s")
        icps = [None] * NSETS
        gcps = [None] * NSETS

        wid = cid * 16 + sid
        u0 = wid * 6
        cnt = jnp.int32(6)

        def offs(step):
            u = u0 + step // NPT
            p = step % NPT
            s = u // NCH
            ch = lax.rem(u, NCH)
            pix0 = ch * CHUNK
            return s, pix0, (s * NPT + p) * P + pix0

        def fetch_idx(step_r, step):
            st = step % NSETS
            gxv, gyv, av = sets[st][8:11]
            semi = sets[st][11]
            _s, _pix0, foff = offs(step_r + step)

            @pl.when(step_r + step < cnt * NPT)
            def _():
                icps[st] = [
                    pltpu.async_copy(gx_hbm.at[pl.ds(foff, CHUNK)], gxv, semi),
                    pltpu.async_copy(gy_hbm.at[pl.ds(foff, CHUNK)], gyv, semi),
                    pltpu.async_copy(attn_hbm.at[pl.ds(foff, CHUNK)], av, semi),
                ]

        def fire(step_r, step):
            st = step % NSETS
            i0, i2 = sets[st][0:2]
            g0, g2 = sets[st][2:4]
            v0, v1, v2, v3 = sets[st][4:8]
            gxv, gyv, av = sets[st][8:11]
            semg = sets[st][12]
            s, _pix0, _foff = offs(step_r + step)
            sbase = s * PP

            @pl.when(step_r + step < cnt * NPT)
            def _():
                for c in icps[st]:
                    c.wait()

                @plsc.parallel_loop(0, CHUNK // 16, 1)
                def dbody(j):
                    sl = pl.ds(j * 16, 16)
                    x1 = gxv[sl] * 96.0 + 0.5
                    y1 = gyv[sl] * 96.0 + 0.5
                    a16 = av[sl]
                    tx = x1.astype(i32)
                    ty = y1.astype(i32)
                    wx1 = x1 - tx.astype(f32)
                    wx0 = 1.0 - wx1
                    wy1 = y1 - ty.astype(f32)
                    wy0 = 1.0 - wy1
                    aw0 = a16 * wy0
                    aw1 = a16 * wy1
                    v0[sl] = aw0 * wx0
                    v1[sl] = aw0 * wx1
                    v2[sl] = aw1 * wx0
                    v3[sl] = aw1 * wx1
                    base = ty * PW + tx + sbase
                    i0[sl] = base
                    i2[sl] = base + PW
                gcps[st] = [
                    pltpu.async_copy(table_hbm.at[i0], g0, semg),
                    pltpu.async_copy(table_hbm.at[i2], g2, semg),
                ]

        def compute(p):
            st = p % NSETS
            g0, g2 = sets[st][2:4]
            v0, v1, v2, v3 = sets[st][4:8]
            first = p == 0

            @plsc.parallel_loop(0, CHUNK // 16, 1)
            def cbody(gi):
                base = gi * 16
                wv0 = v0[pl.ds(base, 16)]
                wv1 = v1[pl.ds(base, 16)]
                wv2 = v2[pl.ds(base, 16)]
                wv3 = v3[pl.ds(base, 16)]
                for i in range(16):
                    px = base + i
                    bw = [_bcast(wv0, i), _bcast(wv1, i),
                          _bcast(wv2, i), _bcast(wv3, i)]
                    xs = [g0[px, pl.ds(0, 16)], g0[px, pl.ds(16, 16)],
                          g2[px, pl.ds(0, 16)], g2[px, pl.ds(16, 16)]]
                    plos, phis = [], []
                    for b, x in zip(bw, xs):
                        xlo = lax.bitcast_convert_type(
                            lax.shift_left(x, 16), f32)
                        xhi = lax.bitcast_convert_type(x, f32)
                        plos.append(b * xlo)
                        phis.append(b * xhi)
                    slo = (plos[0] + plos[1]) + (plos[2] + plos[3])
                    shi = (phis[0] + phis[1]) + (phis[2] + phis[3])
                    if not first:
                        slo = slo + acc[pl.ds(px * HD, 16)]
                        shi = shi + acc[pl.ds(px * HD + 16, 16)]
                    acc[pl.ds(px * HD, 16)] = slo
                    acc[pl.ds(px * HD + 16, 16)] = shi

        def unit_body(ul, carry):
            step_r = ul * NPT
            for p in range(NPT):
                fetch_idx(step_r, p + 2)
                fire(step_r, p + 1)
                for c in gcps[p % NSETS]:
                    c.wait()
                compute(p)
            s, pix0, _f = offs(step_r)
            pltpu.sync_copy(acc, out_hbm.at[pl.ds((s * P + pix0) * HD,
                                                  CHUNK * HD)])
            return carry

        fetch_idx(0, 0)
        fire(0, 0)
        fetch_idx(0, 1)
        lax.fori_loop(0, cnt, unit_body, 0)

    return k(table, attn, gx, gy)


def _proj_body(pw_ref, ws_ref, bias_ref, out_ref):
    h = pl.program_id(1)
    res = lax.dot_general(pw_ref[0], ws_ref[0, 0],
                          (((1,), (1,)), ((), ())),
                          preferred_element_type=jnp.float32)

    @pl.when(h == 0)
    def _():
        out_ref[0] = res + bias_ref[...]

    @pl.when(h > 0)
    def _():
        out_ref[0] = out_ref[0] + res


def _proj(pw2, ws4, pb2):
    B = ws4.shape[0]
    return pl.pallas_call(
        _proj_body,
        grid=(B, NH),
        in_specs=[
            pl.BlockSpec((1, CC, HD), lambda b, h: (h, 0, 0)),
            pl.BlockSpec((1, 1, P, HD), lambda b, h: (b, h, 0, 0)),
            pl.BlockSpec((CC, 1), lambda b, h: (0, 0)),
        ],
        out_specs=pl.BlockSpec((1, CC, P), lambda b, h: (b, 0, 0)),
        out_shape=jax.ShapeDtypeStruct((B, CC, P), jnp.float32),
        compiler_params=pltpu.CompilerParams(
            dimension_semantics=("parallel", "arbitrary")),
    )(pw2, ws4, pb2)


def kernel(query, value, reference_points, attn_conv_w, attn_conv_b,
           proj_w, proj_b):
    B, C, H, W = query.shape

    qpad = jnp.pad(query.astype(jnp.bfloat16), ((0, 0), (0, 0), (1, 1), (1, 1)))
    qext = jnp.pad(qpad.reshape(B, C, PP), ((0, 0), (0, 0), (0, QE - PP)))

    wtap = attn_conv_w.reshape(NG, C, 9).transpose(2, 0, 1) \
                      .astype(jnp.bfloat16)
    bias2 = attn_conv_b.reshape(NG, 1)

    vb = value.astype(jnp.bfloat16).reshape(B, NH, 2, HD // 2, H, W)
    vt = vb.transpose(0, 1, 4, 5, 3, 2)
    vw = lax.bitcast_convert_type(vt, jnp.int32)
    tw = jnp.pad(vw, ((0, 0), (0, 0), (1, 1), (1, 1), (0, 0))) \
            .reshape(B * NH * PP, HD // 2)
    twp = jnp.pad(tw, ((0, 1), (0, 0)))
    table = jnp.concatenate([twp[:-1], twp[1:]], axis=1)

    rpt = reference_points.transpose(0, 3, 4, 5, 1, 2)
    rpp = jnp.pad(rpt, ((0, 0), (0, 0), (0, 0), (0, 0), (0, 2), (0, 2)),
                  constant_values=0.5)
    gx = jnp.pad(rpp[:, :, :, 0].reshape(B, NG, PP),
                 ((0, 0), (0, 0), (0, P - PP)), constant_values=0.5)
    gy = jnp.pad(rpp[:, :, :, 1].reshape(B, NG, PP),
                 ((0, 0), (0, 0), (0, P - PP)), constant_values=0.5)

    attn = _attn_weights(qext, wtap, bias2)

    S = B * NH
    ws = _sc_gather(table, attn.reshape(-1), gx.reshape(-1), gy.reshape(-1),
                    S)

    pw2 = proj_w.reshape(C, NH, HD).transpose(1, 0, 2)
    pb2 = proj_b.reshape(C, 1)
    outflat = _proj(pw2, ws.reshape(B, NH, P, HD), pb2)
    return outflat[:, :, :PP].reshape(B, C, PW, PW)[:, :, :H, :W]
